# baseline (speedup 1.0000x reference)
import functools

import jax
import jax.numpy as jnp
from jax import lax
from jax.experimental import pallas as pl
from jax.experimental.pallas import tpu as pltpu
from jax.experimental.pallas import tpu_sc as plsc

_LANES = 16


def _sc_embed(idx_t, w_flat, *, num_workers):
    jhi, ihi_total, jlo, ilo = idx_t.shape
    ihi_per_w = ihi_total // num_workers

    mesh = plsc.VectorSubcoreMesh(core_axis_name="c", subcore_axis_name="s")

    ihi_c = ihi_per_w // 2

    @functools.partial(
        pl.kernel,
        mesh=mesh,
        out_type=jax.ShapeDtypeStruct(
            (jlo * jhi, ihi_total, 4, ilo), jnp.float32
        ),
        scratch_types=[
            pltpu.VMEM((2, ihi_c, jlo, ilo), jnp.int32),
            pltpu.VMEM((2, jlo, ihi_c, 4, ilo), jnp.float32),
            pltpu.VMEM((32,), jnp.float32),
            pltpu.SemaphoreType.DMA,
            pltpu.SemaphoreType.DMA,
            pltpu.SemaphoreType.DMA,
            pltpu.SemaphoreType.DMA,
        ],
        compiler_params=pltpu.CompilerParams(needs_layout_passes=False),
    )
    def body(idx_hbm, w_hbm, out_hbm, idx_v, out_v, w_v, si0, si1, so0, so1):
        num_cores = lax.axis_size("c")
        wid = lax.axis_index("s") * num_cores + lax.axis_index("c")
        ihi0 = wid * ihi_per_w
        sem_in = (si0, si1)
        sem_out = (so0, so1)

        pltpu.sync_copy(w_hbm, w_v)

        def in_slice(jh, half):
            return idx_hbm.at[jh, pl.ds(ihi0 + half * ihi_c, ihi_c)]

        def out_slice(jh, half):
            return out_hbm.at[
                pl.ds(jh * jlo, jlo), pl.ds(ihi0 + half * ihi_c, ihi_c)
            ]

        pltpu.make_async_copy(in_slice(0, 0), idx_v.at[0], si0).start()

        def jh_body(jh, _):
            for half in (0, 1):
                if half == 0:
                    pltpu.make_async_copy(
                        in_slice(jh, 1), idx_v.at[1], si1
                    ).start()
                else:

                    @pl.when(jh < jhi - 1)
                    def _():
                        pltpu.make_async_copy(
                            in_slice(jh + 1, 0), idx_v.at[0], si0
                        ).start()

                pltpu.make_async_copy(
                    in_slice(jh, half), idx_v.at[half], sem_in[half]
                ).wait()

                @pl.when(jh >= 1)
                def _():
                    pltpu.make_async_copy(
                        out_v.at[half], out_slice(jh, half), sem_out[half]
                    ).wait()

                @plsc.parallel_loop(0, jlo * ihi_c * 8, unroll=64)
                def _(t):
                    jl = lax.shift_right_logical(t, 4)
                    ih = lax.bitwise_and(lax.shift_right_logical(t, 3), 1)
                    v = lax.bitwise_and(t, 7) * _LANES
                    ivec = idx_v[half, ih, jl, pl.ds(v, _LANES)]
                    for c in range(4):
                        out_v[half, jl, ih, c, pl.ds(v, _LANES)] = (
                            plsc.load_gather(w_v.at[pl.ds(c * 8, 8)], [ivec])
                        )
                pltpu.make_async_copy(
                    out_v.at[half], out_slice(jh, half), sem_out[half]
                ).start()
            return 0

        lax.fori_loop(0, jhi, jh_body, 0)
        for half in (0, 1):
            pltpu.make_async_copy(
                out_v.at[half], out_slice(jhi - 1, half), sem_out[half]
            ).wait()

    return body(idx_t, w_flat)


def kernel(indices, weight):
    n_rows, n_cols = indices.shape
    v, d = weight.shape
    jhi, jlo = n_cols // 8, 8
    ihi, ilo = n_rows // 128, 128
    idx_t = (
        indices.astype(jnp.int32)
        .reshape(ihi, ilo, jhi, jlo)
        .transpose(2, 0, 3, 1)
    )
    w_flat = weight.T.reshape(-1)
    o4 = _sc_embed(idx_t, w_flat, num_workers=32)
    return o4.transpose(1, 3, 0, 2).reshape(n_rows, n_cols, d)


# speedup vs baseline: 1.0084x; 1.0084x over previous
import functools

import jax
import jax.numpy as jnp
from jax import lax
from jax.experimental import pallas as pl
from jax.experimental.pallas import tpu as pltpu
from jax.experimental.pallas import tpu_sc as plsc

_LANES = 16


def _sc_embed(idx_t, w_flat, *, num_workers):
    jhi, ihi_total, jlo, ilo = idx_t.shape
    ihi_per_w = ihi_total // num_workers

    mesh = plsc.VectorSubcoreMesh(core_axis_name="c", subcore_axis_name="s")

    ihi_c = ihi_per_w // 2

    @functools.partial(
        pl.kernel,
        mesh=mesh,
        out_type=jax.ShapeDtypeStruct(
            (jlo * jhi, ihi_total, 4, ilo), jnp.float32
        ),
        scratch_types=[
            pltpu.VMEM((2, ihi_c, jlo, ilo), jnp.int32),
            pltpu.VMEM((2, jlo, ihi_c, 4, ilo), jnp.float32),
            pltpu.VMEM((32,), jnp.float32),
            pltpu.SemaphoreType.DMA,
            pltpu.SemaphoreType.DMA,
            pltpu.SemaphoreType.DMA,
            pltpu.SemaphoreType.DMA,
        ],
        compiler_params=pltpu.CompilerParams(needs_layout_passes=False),
    )
    def body(idx_hbm, w_hbm, out_hbm, idx_v, out_v, w_v, si0, si1, so0, so1):
        num_cores = lax.axis_size("c")
        wid = lax.axis_index("s") * num_cores + lax.axis_index("c")
        ihi0 = wid * ihi_per_w
        sem_in = (si0, si1)
        sem_out = (so0, so1)

        pltpu.sync_copy(w_hbm, w_v)

        def in_slice(jh, half):
            return idx_hbm.at[jh, pl.ds(ihi0 + half * ihi_c, ihi_c)]

        def out_slice(jh, half):
            return out_hbm.at[
                pl.ds(jh * jlo, jlo), pl.ds(ihi0 + half * ihi_c, ihi_c)
            ]

        pltpu.make_async_copy(in_slice(0, 0), idx_v.at[0], si0).start()

        def jh_body(jh, _):
            for half in (0, 1):
                if half == 0:
                    pltpu.make_async_copy(
                        in_slice(jh, 1), idx_v.at[1], si1
                    ).start()
                else:

                    @pl.when(jh < jhi - 1)
                    def _():
                        pltpu.make_async_copy(
                            in_slice(jh + 1, 0), idx_v.at[0], si0
                        ).start()

                pltpu.make_async_copy(
                    in_slice(jh, half), idx_v.at[half], sem_in[half]
                ).wait()

                @pl.when(jh >= 1)
                def _():
                    pltpu.make_async_copy(
                        out_v.at[half], out_slice(jh, half), sem_out[half]
                    ).wait()

                vgroups = ilo // _LANES
                vb = vgroups.bit_length() - 1
                ib = ihi_c.bit_length() - 1

                @plsc.parallel_loop(0, jlo * ihi_c * vgroups, unroll=32)
                def _(t):
                    jl = lax.shift_right_logical(t, ib + vb)
                    ih = lax.bitwise_and(
                        lax.shift_right_logical(t, vb), ihi_c - 1
                    )
                    v = lax.bitwise_and(t, vgroups - 1) * _LANES
                    ivec = idx_v[half, ih, jl, pl.ds(v, _LANES)]
                    for c in range(4):
                        out_v[half, jl, ih, c, pl.ds(v, _LANES)] = (
                            plsc.load_gather(w_v.at[pl.ds(c * 8, 8)], [ivec])
                        )
                pltpu.make_async_copy(
                    out_v.at[half], out_slice(jh, half), sem_out[half]
                ).start()
            return 0

        lax.fori_loop(0, jhi, jh_body, 0)
        for half in (0, 1):
            pltpu.make_async_copy(
                out_v.at[half], out_slice(jhi - 1, half), sem_out[half]
            ).wait()

    return body(idx_t, w_flat)


def kernel(indices, weight):
    n_rows, n_cols = indices.shape
    v, d = weight.shape
    jhi, jlo = n_cols // 8, 8
    ihi, ilo = n_rows // 128, 128
    idx_t = (
        indices.astype(jnp.int32)
        .reshape(ihi, ilo, jhi, jlo)
        .transpose(2, 0, 3, 1)
    )
    w_flat = weight.T.reshape(-1)
    o4 = _sc_embed(idx_t, w_flat, num_workers=32)
    return o4.transpose(1, 3, 0, 2).reshape(n_rows, n_cols, d)
